# Initial kernel scaffold; baseline (speedup 1.0000x reference)
import jax
import jax.numpy as jnp
from jax.experimental import pallas as pl


def kernel(distances, edge_index, max_iterations):
    raise NotImplementedError("write your pallas kernel here")


# speedup vs baseline: 50.5206x; 50.5206x over previous
import functools

import jax
import jax.numpy as jnp
from jax import lax
from jax.experimental import pallas as pl
from jax.experimental.pallas import tpu as pltpu
from jax.experimental.pallas import tpu_sc as plsc

N = 100000
NS = 16
NPW = 6272
N_PAD = NS * NPW
E = 1600000
ROW = 128
RPW = 784
E_PAD = NS * RPW * ROW
CH = 112
NMAC = RPW // CH
VPW = NPW // 16


def _mesh():
    return plsc.VectorSubcoreMesh(
        core_axis_name="c", subcore_axis_name="s", num_cores=1
    )


@functools.partial(
    pl.kernel,
    out_type=(
        jax.ShapeDtypeStruct((N_PAD,), jnp.float32),
        jax.ShapeDtypeStruct((2, 16, 16), jnp.float32),
    ),
    mesh=_mesh(),
    scratch_types=[
        pltpu.VMEM_SHARED((N_PAD,), jnp.float32),
        pltpu.VMEM_SHARED((N_PAD,), jnp.float32),
        pltpu.VMEM((NPW,), jnp.float32),
        pltpu.VMEM((NPW,), jnp.float32),
        pltpu.VMEM((NPW,), jnp.float32),
        pltpu.VMEM((CH, ROW), jnp.int32),
        pltpu.VMEM((CH, ROW), jnp.int32),
        pltpu.VMEM((CH, ROW), jnp.float32),
        pltpu.VMEM((16,), jnp.float32),
        pltpu.VMEM((16,), jnp.float32),
    ],
)
def _bfs_step(d_in, src2, dst2, t_in, d_out, flag_out,
              v_sh, cnt_sh, d_vm, w_vm, z_vm, src_vm, dst_vm, val_vm,
              t_vm, f_vm):
    s = lax.axis_index("s")
    nb = s * NPW

    pltpu.sync_copy(d_in.at[pl.ds(nb, NPW)], d_vm)
    pltpu.sync_copy(t_in, t_vm)

    def init_body(j, carry):
        d16 = d_vm[pl.ds(j * 16, 16)]
        w_vm[pl.ds(j * 16, 16)] = jnp.where(d16 < jnp.inf, 1.0, 0.0)
        z_vm[pl.ds(j * 16, 16)] = jnp.zeros((16,), jnp.float32)
        return carry

    lax.fori_loop(0, VPW, init_body, 0)
    pltpu.sync_copy(w_vm, v_sh.at[pl.ds(nb, NPW)])
    pltpu.sync_copy(z_vm, cnt_sh.at[pl.ds(nb, NPW)])
    plsc.subcore_barrier()

    rb = s * RPW

    def macro_body(m, carry):
        r0 = rb + m * CH
        pltpu.sync_copy(src2.at[pl.ds(r0, CH)], src_vm)
        pltpu.sync_copy(dst2.at[pl.ds(r0, CH)], dst_vm)

        def gather_body(j, c):
            pltpu.sync_copy(v_sh.at[src_vm.at[j]], val_vm.at[j])
            return c

        lax.fori_loop(0, CH, gather_body, 0)

        def scatter_body(j, c):
            pltpu.sync_copy(val_vm.at[j], cnt_sh.at[dst_vm.at[j]], add=True)
            return c

        lax.fori_loop(0, CH, scatter_body, 0)
        return carry

    lax.fori_loop(0, NMAC, macro_body, 0)
    plsc.subcore_barrier()

    pltpu.sync_copy(cnt_sh.at[pl.ds(nb, NPW)], w_vm)
    t16 = t_vm[...]

    def upd_body(j, carry):
        inf_acc, new_acc = carry
        d16 = d_vm[pl.ds(j * 16, 16)]
        c16 = w_vm[pl.ds(j * 16, 16)]
        is_inf = d16 == jnp.inf
        newly = jnp.logical_and(is_inf, c16 > 0.0)
        nd16 = jnp.where(newly, t16, d16)
        d_vm[pl.ds(j * 16, 16)] = nd16
        inf_acc = jnp.maximum(inf_acc, jnp.where(nd16 == jnp.inf, 1.0, 0.0))
        new_acc = jnp.maximum(new_acc, jnp.where(newly, 1.0, 0.0))
        return inf_acc, new_acc

    zeros16 = jnp.zeros((16,), jnp.float32)
    inf_acc, new_acc = lax.fori_loop(0, VPW, upd_body, (zeros16, zeros16))
    pltpu.sync_copy(d_vm, d_out.at[pl.ds(nb, NPW)])
    f_vm[...] = inf_acc
    pltpu.sync_copy(f_vm, flag_out.at[0, s])
    f_vm[...] = new_acc
    pltpu.sync_copy(f_vm, flag_out.at[1, s])


def kernel(distances, edge_index, max_iterations):
    src = edge_index[0].astype(jnp.int32)
    dst = edge_index[1].astype(jnp.int32)
    pad_e = E_PAD - E
    src_p = jnp.concatenate([src, jnp.zeros((pad_e,), jnp.int32)])
    dst_p = jnp.concatenate([dst, jnp.full((pad_e,), N, jnp.int32)])
    src2 = src_p.reshape(E_PAD // ROW, ROW)
    dst2 = dst_p.reshape(E_PAD // ROW, ROW)
    d0 = jnp.pad(distances, (0, N_PAD - N))

    def cond_fn(state):
        it, d, flags = state
        inf_any = jnp.max(flags[0]) > 0.0
        new_any = jnp.max(flags[1]) > 0.0
        return jnp.logical_and(jnp.logical_and(inf_any, new_any),
                               it < max_iterations)

    def body_fn(state):
        it, d, flags = state
        t_arr = jnp.broadcast_to((it + 1).astype(jnp.float32), (16,))
        d_new, flags_new = _bfs_step(d, src2, dst2, t_arr)
        return it + 1, d_new, flags_new

    flags0 = jnp.ones((2, 16, 16), jnp.float32)
    _, d_fin, _ = lax.while_loop(cond_fn, body_fn, (jnp.int32(0), d0, flags0))
    return d_fin[:N]


# speedup vs baseline: 105.1968x; 2.0823x over previous
import functools

import jax
import jax.numpy as jnp
from jax import lax
from jax.experimental import pallas as pl
from jax.experimental.pallas import tpu as pltpu
from jax.experimental.pallas import tpu_sc as plsc

N = 100000
NS = 16
NPW = 6272
N_PAD = NS * NPW
E = 1600000
CHE = 14336
NMAC = 7
EPW = CHE * NMAC
E_PAD = NS * EPW
VPW = NPW // 16


def _mesh():
    return plsc.VectorSubcoreMesh(
        core_axis_name="c", subcore_axis_name="s", num_cores=1
    )


@functools.partial(
    pl.kernel,
    out_type=(
        jax.ShapeDtypeStruct((N_PAD,), jnp.float32),
        jax.ShapeDtypeStruct((2, 16, 16), jnp.float32),
    ),
    mesh=_mesh(),
    scratch_types=[
        pltpu.VMEM_SHARED((N_PAD,), jnp.float32),
        pltpu.VMEM_SHARED((N_PAD,), jnp.float32),
        pltpu.VMEM((NPW,), jnp.float32),
        pltpu.VMEM((NPW,), jnp.float32),
        pltpu.VMEM((NPW,), jnp.float32),
        pltpu.VMEM((CHE,), jnp.int32),
        pltpu.VMEM((CHE,), jnp.int32),
        pltpu.VMEM((CHE,), jnp.float32),
        pltpu.VMEM((16,), jnp.float32),
        pltpu.VMEM((16,), jnp.float32),
    ],
)
def _bfs_step(d_in, src2, dst2, t_in, d_out, flag_out,
              v_sh, cnt_sh, d_vm, w_vm, z_vm, src_vm, dst_vm, val_vm,
              t_vm, f_vm):
    s = lax.axis_index("s")
    nb = s * NPW

    pltpu.sync_copy(d_in.at[pl.ds(nb, NPW)], d_vm)
    pltpu.sync_copy(t_in, t_vm)

    def init_body(j, carry):
        d16 = d_vm[pl.ds(j * 16, 16)]
        w_vm[pl.ds(j * 16, 16)] = jnp.where(d16 < jnp.inf, 1.0, 0.0)
        z_vm[pl.ds(j * 16, 16)] = jnp.zeros((16,), jnp.float32)
        return carry

    lax.fori_loop(0, VPW, init_body, 0)
    pltpu.sync_copy(w_vm, v_sh.at[pl.ds(nb, NPW)])
    pltpu.sync_copy(z_vm, cnt_sh.at[pl.ds(nb, NPW)])
    plsc.subcore_barrier()

    eb = s * EPW

    def macro_body(m, carry):
        e0 = eb + m * CHE
        pltpu.sync_copy(src2.at[pl.ds(e0, CHE)], src_vm)
        pltpu.sync_copy(dst2.at[pl.ds(e0, CHE)], dst_vm)
        pltpu.sync_copy(v_sh.at[src_vm], val_vm)
        pltpu.sync_copy(val_vm, cnt_sh.at[dst_vm], add=True)
        return carry

    lax.fori_loop(0, NMAC, macro_body, 0)
    plsc.subcore_barrier()

    pltpu.sync_copy(cnt_sh.at[pl.ds(nb, NPW)], w_vm)
    t16 = t_vm[...]

    def upd_body(j, carry):
        inf_acc, new_acc = carry
        d16 = d_vm[pl.ds(j * 16, 16)]
        c16 = w_vm[pl.ds(j * 16, 16)]
        is_inf = d16 == jnp.inf
        newly = jnp.logical_and(is_inf, c16 > 0.0)
        nd16 = jnp.where(newly, t16, d16)
        d_vm[pl.ds(j * 16, 16)] = nd16
        inf_acc = jnp.maximum(inf_acc, jnp.where(nd16 == jnp.inf, 1.0, 0.0))
        new_acc = jnp.maximum(new_acc, jnp.where(newly, 1.0, 0.0))
        return inf_acc, new_acc

    zeros16 = jnp.zeros((16,), jnp.float32)
    inf_acc, new_acc = lax.fori_loop(0, VPW, upd_body, (zeros16, zeros16))
    pltpu.sync_copy(d_vm, d_out.at[pl.ds(nb, NPW)])
    f_vm[...] = inf_acc
    pltpu.sync_copy(f_vm, flag_out.at[0, s])
    f_vm[...] = new_acc
    pltpu.sync_copy(f_vm, flag_out.at[1, s])


def kernel(distances, edge_index, max_iterations):
    src = edge_index[0].astype(jnp.int32)
    dst = edge_index[1].astype(jnp.int32)
    pad_e = E_PAD - E
    src2 = jnp.concatenate([src, jnp.zeros((pad_e,), jnp.int32)])
    dst2 = jnp.concatenate([dst, jnp.full((pad_e,), N, jnp.int32)])
    d0 = jnp.pad(distances, (0, N_PAD - N))

    def cond_fn(state):
        it, d, flags = state
        inf_any = jnp.max(flags[0]) > 0.0
        new_any = jnp.max(flags[1]) > 0.0
        return jnp.logical_and(jnp.logical_and(inf_any, new_any),
                               it < max_iterations)

    def body_fn(state):
        it, d, flags = state
        t_arr = jnp.broadcast_to((it + 1).astype(jnp.float32), (16,))
        d_new, flags_new = _bfs_step(d, src2, dst2, t_arr)
        return it + 1, d_new, flags_new

    flags0 = jnp.ones((2, 16, 16), jnp.float32)
    _, d_fin, _ = lax.while_loop(cond_fn, body_fn, (jnp.int32(0), d0, flags0))
    return d_fin[:N]


# speedup vs baseline: 109.6136x; 1.0420x over previous
import functools

import jax
import jax.numpy as jnp
from jax import lax
from jax.experimental import pallas as pl
from jax.experimental.pallas import tpu as pltpu
from jax.experimental.pallas import tpu_sc as plsc

N = 100000
NS = 16
NPW = 6272
N_PAD = NS * NPW
E = 1600000
CHE = 14336
NMAC = 7
EPW = CHE * NMAC
E_PAD = NS * EPW
VPW = NPW // 16
K_INNER = 2


def _mesh():
    return plsc.VectorSubcoreMesh(
        core_axis_name="c", subcore_axis_name="s", num_cores=1
    )


@functools.partial(
    pl.kernel,
    out_type=(
        jax.ShapeDtypeStruct((N_PAD,), jnp.float32),
        jax.ShapeDtypeStruct((2, 16, 16), jnp.float32),
    ),
    mesh=_mesh(),
    scratch_types=[
        pltpu.VMEM_SHARED((N_PAD,), jnp.float32),
        pltpu.VMEM_SHARED((N_PAD,), jnp.float32),
        pltpu.VMEM((NPW,), jnp.float32),
        pltpu.VMEM((NPW,), jnp.float32),
        pltpu.VMEM((NPW,), jnp.float32),
        pltpu.VMEM((CHE,), jnp.int32),
        pltpu.VMEM((CHE,), jnp.int32),
        pltpu.VMEM((CHE,), jnp.float32),
        pltpu.VMEM((16,), jnp.float32),
        pltpu.VMEM((16,), jnp.float32),
    ],
)
def _bfs_step(d_in, src2, dst2, t_in, d_out, flag_out,
              v_sh, cnt_sh, d_vm, w_vm, z_vm, src_vm, dst_vm, val_vm,
              t_vm, f_vm):
    s = lax.axis_index("s")
    nb = s * NPW
    eb = s * EPW

    pltpu.sync_copy(d_in.at[pl.ds(nb, NPW)], d_vm)
    pltpu.sync_copy(t_in, t_vm)

    def one_iter(k, carry):
        def init_body(j, c):
            d16 = d_vm[pl.ds(j * 16, 16)]
            w_vm[pl.ds(j * 16, 16)] = jnp.where(d16 < jnp.inf, 1.0, 0.0)
            z_vm[pl.ds(j * 16, 16)] = jnp.zeros((16,), jnp.float32)
            return c

        lax.fori_loop(0, VPW, init_body, 0)
        pltpu.sync_copy(w_vm, v_sh.at[pl.ds(nb, NPW)])
        pltpu.sync_copy(z_vm, cnt_sh.at[pl.ds(nb, NPW)])
        plsc.subcore_barrier()

        def macro_body(m, c):
            e0 = eb + m * CHE
            pltpu.sync_copy(src2.at[pl.ds(e0, CHE)], src_vm)
            pltpu.sync_copy(dst2.at[pl.ds(e0, CHE)], dst_vm)
            pltpu.sync_copy(v_sh.at[src_vm], val_vm)
            pltpu.sync_copy(val_vm, cnt_sh.at[dst_vm], add=True)
            return c

        lax.fori_loop(0, NMAC, macro_body, 0)
        plsc.subcore_barrier()

        pltpu.sync_copy(cnt_sh.at[pl.ds(nb, NPW)], w_vm)
        t16 = t_vm[...] + lax.convert_element_type(k, jnp.float32)

        def upd_body(j, c):
            inf_acc, new_acc = c
            d16 = d_vm[pl.ds(j * 16, 16)]
            c16 = w_vm[pl.ds(j * 16, 16)]
            is_inf = d16 == jnp.inf
            newly = jnp.logical_and(is_inf, c16 > 0.0)
            nd16 = jnp.where(newly, t16, d16)
            d_vm[pl.ds(j * 16, 16)] = nd16
            inf_acc = jnp.maximum(inf_acc, jnp.where(nd16 == jnp.inf, 1.0, 0.0))
            new_acc = jnp.maximum(new_acc, jnp.where(newly, 1.0, 0.0))
            return inf_acc, new_acc

        zeros16 = jnp.zeros((16,), jnp.float32)
        inf_acc, new_acc = lax.fori_loop(0, VPW, upd_body, (zeros16, zeros16))
        f_vm[...] = inf_acc
        pltpu.sync_copy(f_vm, flag_out.at[0, s])
        f_vm[...] = new_acc
        pltpu.sync_copy(f_vm, flag_out.at[1, s])
        plsc.subcore_barrier()
        return carry

    lax.fori_loop(0, K_INNER, one_iter, 0)
    pltpu.sync_copy(d_vm, d_out.at[pl.ds(nb, NPW)])


def kernel(distances, edge_index, max_iterations):
    src = edge_index[0].astype(jnp.int32)
    dst = edge_index[1].astype(jnp.int32)
    pad_e = E_PAD - E
    src2 = jnp.concatenate([src, jnp.zeros((pad_e,), jnp.int32)])
    dst2 = jnp.concatenate([dst, jnp.full((pad_e,), N, jnp.int32)])
    d0 = jnp.pad(distances, (0, N_PAD - N))

    def cond_fn(state):
        it, d, flags = state
        inf_any = jnp.max(flags[0]) > 0.0
        new_any = jnp.max(flags[1]) > 0.0
        return jnp.logical_and(jnp.logical_and(inf_any, new_any),
                               it < max_iterations)

    def body_fn(state):
        it, d, flags = state
        t_arr = jnp.broadcast_to((it + 1).astype(jnp.float32), (16,))
        d_new, flags_new = _bfs_step(d, src2, dst2, t_arr)
        return it + K_INNER, d_new, flags_new

    flags0 = jnp.ones((2, 16, 16), jnp.float32)
    _, d_fin, _ = lax.while_loop(cond_fn, body_fn, (jnp.int32(0), d0, flags0))
    return d_fin[:N]


# speedup vs baseline: 121.3803x; 1.1073x over previous
import functools

import jax
import jax.numpy as jnp
from jax import lax
from jax.experimental import pallas as pl
from jax.experimental.pallas import tpu as pltpu
from jax.experimental.pallas import tpu_sc as plsc

N = 100000
NS = 16
NPW = 6272
N_PAD = NS * NPW
E = 1600000
CHE = 14336
NMAC = 7
EPW = CHE * NMAC
E_PAD = NS * EPW
VPW = NPW // 16
K_INNER = 2


def _mesh():
    return plsc.VectorSubcoreMesh(
        core_axis_name="c", subcore_axis_name="s", num_cores=1
    )


@functools.partial(
    pl.kernel,
    out_type=(
        jax.ShapeDtypeStruct((N_PAD,), jnp.float32),
        jax.ShapeDtypeStruct((2, 16, 16), jnp.float32),
    ),
    mesh=_mesh(),
    scratch_types=[
        pltpu.VMEM_SHARED((N_PAD,), jnp.float32),
        pltpu.VMEM_SHARED((N_PAD,), jnp.float32),
        pltpu.VMEM((NPW,), jnp.float32),
        pltpu.VMEM((NPW,), jnp.float32),
        pltpu.VMEM((NPW,), jnp.float32),
        pltpu.VMEM((CHE,), jnp.int32),
        pltpu.VMEM((CHE,), jnp.int32),
        pltpu.VMEM((CHE,), jnp.int32),
        pltpu.VMEM((CHE,), jnp.int32),
        pltpu.VMEM((CHE,), jnp.float32),
        pltpu.VMEM((16,), jnp.float32),
        pltpu.VMEM((16,), jnp.float32),
        pltpu.SemaphoreType.DMA,
        pltpu.SemaphoreType.DMA,
    ],
)
def _bfs_step(d_in, src2, dst2, t_in, d_out, flag_out,
              v_sh, cnt_sh, d_vm, w_vm, z_vm, src_vm0, dst_vm0, src_vm1,
              dst_vm1, val_vm, t_vm, f_vm, sem0, sem1):
    s = lax.axis_index("s")
    nb = s * NPW
    eb = s * EPW

    pltpu.sync_copy(d_in.at[pl.ds(nb, NPW)], d_vm)
    pltpu.sync_copy(t_in, t_vm)

    def one_iter(k, carry):
        def init_body(j, c):
            d16 = d_vm[pl.ds(j * 16, 16)]
            w_vm[pl.ds(j * 16, 16)] = jnp.where(d16 < jnp.inf, 1.0, 0.0)
            z_vm[pl.ds(j * 16, 16)] = jnp.zeros((16,), jnp.float32)
            return c

        lax.fori_loop(0, VPW, init_body, 0)
        pltpu.sync_copy(w_vm, v_sh.at[pl.ds(nb, NPW)])
        pltpu.sync_copy(z_vm, cnt_sh.at[pl.ds(nb, NPW)])
        plsc.subcore_barrier()

        bufs = ((src_vm0, dst_vm0, sem0), (src_vm1, dst_vm1, sem1))
        pend = pltpu.async_copy(src2.at[pl.ds(eb, CHE)], src_vm0, sem0)
        pend2 = pltpu.async_copy(dst2.at[pl.ds(eb, CHE)], dst_vm0, sem0)
        for m in range(NMAC):
            sv, dv, _ = bufs[m % 2]
            if m + 1 < NMAC:
                nsv, ndv, nsem = bufs[(m + 1) % 2]
                e1 = eb + (m + 1) * CHE
                nxt = pltpu.async_copy(src2.at[pl.ds(e1, CHE)], nsv, nsem)
                nxt2 = pltpu.async_copy(dst2.at[pl.ds(e1, CHE)], ndv, nsem)
            pend.wait()
            pend2.wait()
            pltpu.sync_copy(v_sh.at[sv], val_vm)
            pltpu.sync_copy(val_vm, cnt_sh.at[dv], add=True)
            if m + 1 < NMAC:
                pend, pend2 = nxt, nxt2
        plsc.subcore_barrier()

        pltpu.sync_copy(cnt_sh.at[pl.ds(nb, NPW)], w_vm)
        t16 = t_vm[...] + lax.convert_element_type(k, jnp.float32)

        def upd_body(j, c):
            inf_acc, new_acc = c
            d16 = d_vm[pl.ds(j * 16, 16)]
            c16 = w_vm[pl.ds(j * 16, 16)]
            is_inf = d16 == jnp.inf
            newly = jnp.logical_and(is_inf, c16 > 0.0)
            nd16 = jnp.where(newly, t16, d16)
            d_vm[pl.ds(j * 16, 16)] = nd16
            inf_acc = jnp.maximum(inf_acc, jnp.where(nd16 == jnp.inf, 1.0, 0.0))
            new_acc = jnp.maximum(new_acc, jnp.where(newly, 1.0, 0.0))
            return inf_acc, new_acc

        zeros16 = jnp.zeros((16,), jnp.float32)
        inf_acc, new_acc = lax.fori_loop(0, VPW, upd_body, (zeros16, zeros16))
        f_vm[...] = inf_acc
        pltpu.sync_copy(f_vm, flag_out.at[0, s])
        f_vm[...] = new_acc
        pltpu.sync_copy(f_vm, flag_out.at[1, s])
        plsc.subcore_barrier()
        return carry

    lax.fori_loop(0, K_INNER, one_iter, 0)
    pltpu.sync_copy(d_vm, d_out.at[pl.ds(nb, NPW)])


def kernel(distances, edge_index, max_iterations):
    src = edge_index[0].astype(jnp.int32)
    dst = edge_index[1].astype(jnp.int32)
    pad_e = E_PAD - E
    src2 = jnp.concatenate([src, jnp.zeros((pad_e,), jnp.int32)])
    dst2 = jnp.concatenate([dst, jnp.full((pad_e,), N, jnp.int32)])
    d0 = jnp.pad(distances, (0, N_PAD - N))

    def cond_fn(state):
        it, d, flags = state
        inf_any = jnp.max(flags[0]) > 0.0
        new_any = jnp.max(flags[1]) > 0.0
        return jnp.logical_and(jnp.logical_and(inf_any, new_any),
                               it < max_iterations)

    def body_fn(state):
        it, d, flags = state
        t_arr = jnp.broadcast_to((it + 1).astype(jnp.float32), (16,))
        d_new, flags_new = _bfs_step(d, src2, dst2, t_arr)
        return it + K_INNER, d_new, flags_new

    flags0 = jnp.ones((2, 16, 16), jnp.float32)
    _, d_fin, _ = lax.while_loop(cond_fn, body_fn, (jnp.int32(0), d0, flags0))
    return d_fin[:N]


# speedup vs baseline: 122.9804x; 1.0132x over previous
import functools

import jax
import jax.numpy as jnp
from jax import lax
from jax.experimental import pallas as pl
from jax.experimental.pallas import tpu as pltpu
from jax.experimental.pallas import tpu_sc as plsc

N = 100000
NS = 16
NPW = 6272
N_PAD = NS * NPW
E = 1600000
CHE = 14336
NMAC = 7
EPW = CHE * NMAC
E_PAD = NS * EPW
VPW = NPW // 16
K_INNER = 3


def _mesh():
    return plsc.VectorSubcoreMesh(
        core_axis_name="c", subcore_axis_name="s", num_cores=1
    )


@functools.partial(
    pl.kernel,
    out_type=(
        jax.ShapeDtypeStruct((N_PAD,), jnp.float32),
        jax.ShapeDtypeStruct((2, 16, 16), jnp.float32),
    ),
    mesh=_mesh(),
    scratch_types=[
        pltpu.VMEM_SHARED((N_PAD,), jnp.float32),
        pltpu.VMEM_SHARED((N_PAD,), jnp.float32),
        pltpu.VMEM((NPW,), jnp.float32),
        pltpu.VMEM((NPW,), jnp.float32),
        pltpu.VMEM((NPW,), jnp.float32),
        pltpu.VMEM((CHE,), jnp.int32),
        pltpu.VMEM((CHE,), jnp.int32),
        pltpu.VMEM((CHE,), jnp.int32),
        pltpu.VMEM((CHE,), jnp.int32),
        pltpu.VMEM((CHE,), jnp.float32),
        pltpu.VMEM((16,), jnp.float32),
        pltpu.VMEM((16,), jnp.float32),
        pltpu.VMEM((16,), jnp.float32),
        pltpu.SemaphoreType.DMA,
        pltpu.SemaphoreType.DMA,
    ],
)
def _bfs_step(d_in, src2, dst2, t_in, m_in, d_out, flag_out,
              v_sh, cnt_sh, d_vm, w_vm, z_vm, src_vm0, dst_vm0, src_vm1,
              dst_vm1, val_vm, t_vm, m_vm, f_vm, sem0, sem1):
    s = lax.axis_index("s")
    nb = s * NPW
    eb = s * EPW

    pltpu.sync_copy(d_in.at[pl.ds(nb, NPW)], d_vm)
    pltpu.sync_copy(t_in, t_vm)
    pltpu.sync_copy(m_in, m_vm)

    def one_iter(k, carry):
        def init_body(j, c):
            d16 = d_vm[pl.ds(j * 16, 16)]
            w_vm[pl.ds(j * 16, 16)] = jnp.where(d16 < jnp.inf, 1.0, 0.0)
            z_vm[pl.ds(j * 16, 16)] = jnp.zeros((16,), jnp.float32)
            return c

        lax.fori_loop(0, VPW, init_body, 0)
        pltpu.sync_copy(w_vm, v_sh.at[pl.ds(nb, NPW)])
        pltpu.sync_copy(z_vm, cnt_sh.at[pl.ds(nb, NPW)])
        plsc.subcore_barrier()

        bufs = ((src_vm0, dst_vm0, sem0), (src_vm1, dst_vm1, sem1))
        pend = pltpu.async_copy(src2.at[pl.ds(eb, CHE)], src_vm0, sem0)
        pend2 = pltpu.async_copy(dst2.at[pl.ds(eb, CHE)], dst_vm0, sem0)
        for m in range(NMAC):
            sv, dv, _ = bufs[m % 2]
            if m + 1 < NMAC:
                nsv, ndv, nsem = bufs[(m + 1) % 2]
                e1 = eb + (m + 1) * CHE
                nxt = pltpu.async_copy(src2.at[pl.ds(e1, CHE)], nsv, nsem)
                nxt2 = pltpu.async_copy(dst2.at[pl.ds(e1, CHE)], ndv, nsem)
            pend.wait()
            pend2.wait()
            pltpu.sync_copy(v_sh.at[sv], val_vm)
            pltpu.sync_copy(val_vm, cnt_sh.at[dv], add=True)
            if m + 1 < NMAC:
                pend, pend2 = nxt, nxt2
        plsc.subcore_barrier()

        pltpu.sync_copy(cnt_sh.at[pl.ds(nb, NPW)], w_vm)
        t16 = t_vm[...] + lax.convert_element_type(k, jnp.float32)
        in_range = t16 <= m_vm[...]

        def upd_body(j, c):
            inf_acc, new_acc = c
            d16 = d_vm[pl.ds(j * 16, 16)]
            c16 = w_vm[pl.ds(j * 16, 16)]
            is_inf = d16 == jnp.inf
            newly = jnp.logical_and(jnp.logical_and(is_inf, c16 > 0.0),
                                    in_range)
            nd16 = jnp.where(newly, t16, d16)
            d_vm[pl.ds(j * 16, 16)] = nd16
            inf_acc = jnp.maximum(inf_acc, jnp.where(nd16 == jnp.inf, 1.0, 0.0))
            new_acc = jnp.maximum(new_acc, jnp.where(newly, 1.0, 0.0))
            return inf_acc, new_acc

        zeros16 = jnp.zeros((16,), jnp.float32)
        inf_acc, new_acc = lax.fori_loop(0, VPW, upd_body, (zeros16, zeros16))
        f_vm[...] = inf_acc
        pltpu.sync_copy(f_vm, flag_out.at[0, s])
        f_vm[...] = new_acc
        pltpu.sync_copy(f_vm, flag_out.at[1, s])
        plsc.subcore_barrier()
        return carry

    lax.fori_loop(0, K_INNER, one_iter, 0)
    pltpu.sync_copy(d_vm, d_out.at[pl.ds(nb, NPW)])


def kernel(distances, edge_index, max_iterations):
    src = edge_index[0].astype(jnp.int32)
    dst = edge_index[1].astype(jnp.int32)
    pad_e = E_PAD - E
    src2 = jnp.concatenate([src, jnp.zeros((pad_e,), jnp.int32)])
    dst2 = jnp.concatenate([dst, jnp.full((pad_e,), N, jnp.int32)])
    d0 = jnp.pad(distances, (0, N_PAD - N))

    m_arr = jnp.broadcast_to(jnp.float32(max_iterations), (16,))

    def cond_fn(state):
        it, d, flags = state
        inf_any = jnp.max(flags[0]) > 0.0
        new_any = jnp.max(flags[1]) > 0.0
        return jnp.logical_and(jnp.logical_and(inf_any, new_any),
                               it < max_iterations)

    def body_fn(state):
        it, d, flags = state
        t_arr = jnp.broadcast_to((it + 1).astype(jnp.float32), (16,))
        d_new, flags_new = _bfs_step(d, src2, dst2, t_arr, m_arr)
        return it + K_INNER, d_new, flags_new

    flags0 = jnp.ones((2, 16, 16), jnp.float32)
    _, d_fin, _ = lax.while_loop(cond_fn, body_fn, (jnp.int32(0), d0, flags0))
    return d_fin[:N]
